# baseline (speedup 1.0000x reference)
import functools

import jax
import jax.numpy as jnp
from jax.experimental import pallas as pl
from jax.experimental.pallas import tpu as pltpu


def _phase_image(u):
    ci, h, wo = u.shape
    ho = h // 2
    f32 = lambda v: jax.lax.bitcast_convert_type(v, jnp.float32)
    u4 = u.reshape(ci, ho, 2, wo)
    u0, u1 = u4[:, :, 0, :], u4[:, :, 1, :]
    r0c0, r0c1 = f32(u0 << 16), f32(u0 & jnp.uint32(0xFFFF0000))
    r1c0, r1c1 = f32(u1 << 16), f32(u1 & jnp.uint32(0xFFFF0000))
    pz = lambda a, dr, dc: jnp.pad(
        a, ((0, 1), (dr, 2 - dr), (dc, 128 - wo - dc)))
    buf = jnp.concatenate([
        pz(r1c1, 1, 1),
        pz(r1c0, 1, 0),
        pz(r0c1, 0, 1),
        pz(r0c0, 0, 0),
    ], axis=0)
    return buf.reshape(16, (ho + 2) * 128)


def _conv_image(u, w):
    flat = _phase_image(u)
    m = (flat.shape[1] // 128 - 2) * 128
    parts = [flat[:, ah * 128 + aw: ah * 128 + aw + m]
             for ah in (0, 1) for aw in (0, 1)]
    p = jnp.concatenate(parts, axis=0)
    return jnp.dot(w, p, preferred_element_type=jnp.float32)


def _fused_kernel(xu_ref, w_ref, g_ref, bt_ref, o_ref,
                  stat_vmem, sc_vmem, sh_vmem, *, ho, wo, imgs, inv_count, eps):
    phase = pl.program_id(0)
    i = pl.program_id(1)
    n_i = pl.num_programs(1)

    @pl.when(phase == 0)
    def _():
        @pl.when(i == 0)
        def _():
            stat_vmem[...] = jnp.zeros_like(stat_vmem)

        u_all = xu_ref[...].reshape(imgs, -1, 2 * ho, xu_ref.shape[-1])
        for img in range(imgs):
            y = _conv_image(u_all[img], w_ref[...])
            y3 = y.reshape(y.shape[0], ho, 128)
            stat_vmem[0] += jnp.sum(y3, axis=1)
            stat_vmem[1] += jnp.sum(y3 * y3, axis=1)

        @pl.when(i == n_i - 1)
        def _():
            lane = jax.lax.broadcasted_iota(jnp.int32, (1, 128), 1)
            maskf = (lane < wo).astype(jnp.float32)
            s = jnp.sum(stat_vmem[0] * maskf, axis=1, keepdims=True)
            q = jnp.sum(stat_vmem[1] * maskf, axis=1, keepdims=True)
            mean = s * inv_count
            var = q * inv_count - mean * mean
            sc = g_ref[...] * jax.lax.rsqrt(var + eps)
            sc_vmem[...] = sc
            sh_vmem[...] = bt_ref[...] - mean * sc

    @pl.when(phase == 1)
    def _():
        sc = sc_vmem[...]
        sh = sh_vmem[...]
        u_all = xu_ref[...].reshape(imgs, -1, 2 * ho, xu_ref.shape[-1])
        for img in range(imgs):
            y = _conv_image(u_all[img], w_ref[...])
            z = jnp.maximum(y * sc + sh, 0.0)
            o_ref[img] = z.reshape(z.shape[0], ho, 128)[:, :, :wo]


def kernel(x, w, b, gamma, beta):
    del b
    eps = 1e-5
    N, C_in, H, W = x.shape
    C_out = w.shape[0]
    Ho, Wo = H // 2, W // 2
    M = N * Ho * Wo

    x16 = x.astype(jnp.bfloat16).reshape(N, C_in, H, Wo, 2)
    xu = jax.lax.bitcast_convert_type(x16, jnp.uint32)
    xu = xu.reshape(N * C_in * H, Wo)

    w6 = w.reshape(C_out, C_in, 2, 2, 2, 2)
    w2 = w6.transpose(0, 2, 4, 3, 5, 1)
    w2 = jnp.pad(w2, ((0, 0),) * 5 + ((0, 4 - C_in),))
    w2 = w2.reshape(C_out, 64)

    IMG = 2 if N % 2 == 0 else 1
    vmem_limit = 100 << 20

    out = pl.pallas_call(
        functools.partial(_fused_kernel, ho=Ho, wo=Wo, imgs=IMG,
                          inv_count=float(1.0 / M), eps=float(eps)),
        out_shape=jax.ShapeDtypeStruct((N, C_out, Ho, Wo), jnp.float32),
        grid_spec=pltpu.PrefetchScalarGridSpec(
            num_scalar_prefetch=0,
            grid=(2, N // IMG),
            in_specs=[
                pl.BlockSpec((IMG * C_in * H, Wo), lambda p, i: (i, 0)),
                pl.BlockSpec((C_out, 64), lambda p, i: (0, 0)),
                pl.BlockSpec((C_out, 1), lambda p, i: (0, 0)),
                pl.BlockSpec((C_out, 1), lambda p, i: (0, 0)),
            ],
            out_specs=pl.BlockSpec((IMG, C_out, Ho, Wo),
                                   lambda p, i: (i * p, 0, 0, 0)),
            scratch_shapes=[pltpu.VMEM((2, C_out, 128), jnp.float32),
                            pltpu.VMEM((C_out, 1), jnp.float32),
                            pltpu.VMEM((C_out, 1), jnp.float32)]),
        compiler_params=pltpu.CompilerParams(
            dimension_semantics=("arbitrary", "arbitrary"),
            vmem_limit_bytes=vmem_limit),
    )(xu, w2, gamma.astype(jnp.float32).reshape(C_out, 1),
      beta.astype(jnp.float32).reshape(C_out, 1))

    return out


# speedup vs baseline: 1.0277x; 1.0277x over previous
import functools

import jax
import jax.numpy as jnp
from jax.experimental import pallas as pl
from jax.experimental.pallas import tpu as pltpu


def _phase_image(u):
    ci, h, wo = u.shape
    ho = h // 2
    f32 = lambda v: jax.lax.bitcast_convert_type(v, jnp.float32)
    u4 = u.reshape(ci, ho, 2, wo)
    u0, u1 = u4[:, :, 0, :], u4[:, :, 1, :]
    r0c0, r0c1 = f32(u0 << 16), f32(u0 & jnp.uint32(0xFFFF0000))
    r1c0, r1c1 = f32(u1 << 16), f32(u1 & jnp.uint32(0xFFFF0000))
    pz = lambda a, dr, dc: jnp.pad(
        a, ((0, 1), (dr, 2 - dr), (dc, 128 - wo - dc)))
    buf = jnp.concatenate([
        pz(r1c1, 1, 1),
        pz(r1c0, 1, 0),
        pz(r0c1, 0, 1),
        pz(r0c0, 0, 0),
    ], axis=0)
    return buf.reshape(16, (ho + 2) * 128)


def _conv_image(u, w):
    flat = _phase_image(u)
    m = (flat.shape[1] // 128 - 2) * 128
    parts = [flat[:, ah * 128 + aw: ah * 128 + aw + m]
             for ah in (0, 1) for aw in (0, 1)]
    p = jnp.concatenate(parts, axis=0)
    return jnp.dot(w, p, preferred_element_type=jnp.float32)


def _fused_kernel(xu_ref, w_ref, g_ref, bt_ref, o_ref,
                  stat_vmem, sc_vmem, sh_vmem, *, ho, wo, imgs, inv_count, eps):
    phase = pl.program_id(0)
    i = pl.program_id(1)
    n_i = pl.num_programs(1)

    @pl.when(phase == 0)
    def _():
        @pl.when(i == 0)
        def _():
            stat_vmem[...] = jnp.zeros_like(stat_vmem)

        lane = jax.lax.broadcasted_iota(jnp.int32, (ho * 128, 1), 0)
        onesm = ((lane & 127) < wo).astype(jnp.float32)
        for img in range(imgs):
            y = _conv_image(xu_ref[img], w_ref[...])
            stat_vmem[0, :, 0:1] += jnp.dot(
                y, onesm, preferred_element_type=jnp.float32)
            stat_vmem[1, :, 0:1] += jnp.dot(
                y * y, onesm, preferred_element_type=jnp.float32)

        @pl.when(i == n_i - 1)
        def _():
            s = stat_vmem[0, :, 0:1]
            q = stat_vmem[1, :, 0:1]
            mean = s * inv_count
            var = q * inv_count - mean * mean
            sc = g_ref[...] * jax.lax.rsqrt(var + eps)
            sc_vmem[...] = sc
            sh_vmem[...] = bt_ref[...] - mean * sc

    @pl.when(phase == 1)
    def _():
        sc = sc_vmem[...]
        sh = sh_vmem[...]
        for img in range(imgs):
            y = _conv_image(xu_ref[img], w_ref[...])
            z = jnp.maximum(y * sc + sh, 0.0)
            o_ref[img] = z.reshape(z.shape[0], ho, 128)[:, :, :wo]


def kernel(x, w, b, gamma, beta):
    del b
    eps = 1e-5
    N, C_in, H, W = x.shape
    C_out = w.shape[0]
    Ho, Wo = H // 2, W // 2
    M = N * Ho * Wo

    x16 = x.astype(jnp.bfloat16).reshape(N, C_in, H, Wo, 2)
    xu = jax.lax.bitcast_convert_type(x16, jnp.uint32)

    w6 = w.reshape(C_out, C_in, 2, 2, 2, 2)
    w2 = w6.transpose(0, 2, 4, 3, 5, 1)
    w2 = jnp.pad(w2, ((0, 0),) * 5 + ((0, 4 - C_in),))
    w2 = w2.reshape(C_out, 64)

    IMG = 2 if N % 2 == 0 else 1
    vmem_limit = 100 << 20

    out = pl.pallas_call(
        functools.partial(_fused_kernel, ho=Ho, wo=Wo, imgs=IMG,
                          inv_count=float(1.0 / M), eps=float(eps)),
        out_shape=jax.ShapeDtypeStruct((N, C_out, Ho, Wo), jnp.float32),
        grid_spec=pltpu.PrefetchScalarGridSpec(
            num_scalar_prefetch=0,
            grid=(2, N // IMG),
            in_specs=[
                pl.BlockSpec((IMG, C_in, H, Wo), lambda p, i: (i, 0, 0, 0)),
                pl.BlockSpec((C_out, 64), lambda p, i: (0, 0)),
                pl.BlockSpec((C_out, 1), lambda p, i: (0, 0)),
                pl.BlockSpec((C_out, 1), lambda p, i: (0, 0)),
            ],
            out_specs=pl.BlockSpec((IMG, C_out, Ho, Wo),
                                   lambda p, i: (i * p, 0, 0, 0)),
            scratch_shapes=[pltpu.VMEM((2, C_out, 128), jnp.float32),
                            pltpu.VMEM((C_out, 1), jnp.float32),
                            pltpu.VMEM((C_out, 1), jnp.float32)]),
        compiler_params=pltpu.CompilerParams(
            dimension_semantics=("arbitrary", "arbitrary"),
            vmem_limit_bytes=vmem_limit),
    )(xu, w2, gamma.astype(jnp.float32).reshape(C_out, 1),
      beta.astype(jnp.float32).reshape(C_out, 1))

    return out


# speedup vs baseline: 1.1904x; 1.1583x over previous
import functools

import jax
import jax.numpy as jnp
from jax.experimental import pallas as pl
from jax.experimental.pallas import tpu as pltpu


def _phase_image(u):
    ci, h, wo = u.shape
    ho = h // 2
    f32 = lambda v: jax.lax.bitcast_convert_type(v, jnp.float32)
    u4 = u.reshape(ci, ho, 2, wo)
    u0, u1 = u4[:, :, 0, :], u4[:, :, 1, :]
    r0c0, r0c1 = f32(u0 << 16), f32(u0 & jnp.uint32(0xFFFF0000))
    r1c0, r1c1 = f32(u1 << 16), f32(u1 & jnp.uint32(0xFFFF0000))
    pz = lambda a, dr, dc: jnp.pad(
        a, ((0, 1), (dr, 2 - dr), (dc, 128 - wo - dc)))
    buf = jnp.concatenate([
        pz(r1c1, 1, 1),
        pz(r1c0, 1, 0),
        pz(r0c1, 0, 1),
        pz(r0c0, 0, 0),
    ], axis=0)
    return buf.reshape(16, (ho + 2) * 128)


def _conv_image(u, w):
    flat = _phase_image(u)
    m = (flat.shape[1] // 128 - 2) * 128
    parts = [flat[:, ah * 128 + aw: ah * 128 + aw + m]
             for ah in (0, 1) for aw in (0, 1)]
    p = jnp.concatenate(parts, axis=0)
    return jnp.dot(w, p, preferred_element_type=jnp.float32)


def _fused_kernel(xu_ref, w_ref, g_ref, bt_ref, o_ref,
                  stat_vmem, sc_vmem, sh_vmem, *, ho, wo, imgs, inv_count, eps):
    phase = pl.program_id(0)
    i = pl.program_id(1)
    n_i = pl.num_programs(1)

    @pl.when(phase == 0)
    def _():
        @pl.when(i == 0)
        def _():
            stat_vmem[...] = jnp.zeros_like(stat_vmem)

        for img in range(imgs):
            y = _conv_image(xu_ref[img], w_ref[...])
            y3 = y.reshape(y.shape[0], ho, 128)
            stat_vmem[0] += jnp.sum(y3, axis=1)
            stat_vmem[1] += jnp.sum(y3 * y3, axis=1)

        @pl.when(i == n_i - 1)
        def _():
            lane = jax.lax.broadcasted_iota(jnp.int32, (1, 128), 1)
            maskf = (lane < wo).astype(jnp.float32)
            s = jnp.sum(stat_vmem[0] * maskf, axis=1, keepdims=True)
            q = jnp.sum(stat_vmem[1] * maskf, axis=1, keepdims=True)
            mean = s * inv_count
            var = q * inv_count - mean * mean
            sc = g_ref[...] * jax.lax.rsqrt(var + eps)
            sc_vmem[...] = sc
            sh_vmem[...] = bt_ref[...] - mean * sc

    @pl.when(phase == 1)
    def _():
        sc = sc_vmem[...]
        sh = sh_vmem[...]
        for img in range(imgs):
            y = _conv_image(xu_ref[img], w_ref[...])
            z = jnp.maximum(y * sc + sh, 0.0)
            o_ref[img] = z.reshape(z.shape[0], ho, 128)[:, :, :wo]


def kernel(x, w, b, gamma, beta):
    del b
    eps = 1e-5
    N, C_in, H, W = x.shape
    C_out = w.shape[0]
    Ho, Wo = H // 2, W // 2
    M = N * Ho * Wo

    x16 = x.astype(jnp.bfloat16).reshape(N, C_in, H, Wo, 2)
    xu = jax.lax.bitcast_convert_type(x16, jnp.uint32)

    w6 = w.reshape(C_out, C_in, 2, 2, 2, 2)
    w2 = w6.transpose(0, 2, 4, 3, 5, 1)
    w2 = jnp.pad(w2, ((0, 0),) * 5 + ((0, 4 - C_in),))
    w2 = w2.reshape(C_out, 64)

    IMG = 2 if N % 2 == 0 else 1
    vmem_limit = 100 << 20

    out = pl.pallas_call(
        functools.partial(_fused_kernel, ho=Ho, wo=Wo, imgs=IMG,
                          inv_count=float(1.0 / M), eps=float(eps)),
        out_shape=jax.ShapeDtypeStruct((N, C_out, Ho, Wo), jnp.float32),
        grid_spec=pltpu.PrefetchScalarGridSpec(
            num_scalar_prefetch=0,
            grid=(2, N // IMG),
            in_specs=[
                pl.BlockSpec((IMG, C_in, H, Wo), lambda p, i: (i, 0, 0, 0)),
                pl.BlockSpec((C_out, 64), lambda p, i: (0, 0)),
                pl.BlockSpec((C_out, 1), lambda p, i: (0, 0)),
                pl.BlockSpec((C_out, 1), lambda p, i: (0, 0)),
            ],
            out_specs=pl.BlockSpec((IMG, C_out, Ho, Wo),
                                   lambda p, i: (i * p, 0, 0, 0)),
            scratch_shapes=[pltpu.VMEM((2, C_out, 128), jnp.float32),
                            pltpu.VMEM((C_out, 1), jnp.float32),
                            pltpu.VMEM((C_out, 1), jnp.float32)]),
        compiler_params=pltpu.CompilerParams(
            dimension_semantics=("arbitrary", "arbitrary"),
            vmem_limit_bytes=vmem_limit),
    )(xu, w2, gamma.astype(jnp.float32).reshape(C_out, 1),
      beta.astype(jnp.float32).reshape(C_out, 1))

    return out


# speedup vs baseline: 1.3346x; 1.1212x over previous
import functools

import jax
import jax.numpy as jnp
from jax.experimental import pallas as pl
from jax.experimental.pallas import tpu as pltpu


def _phase_image(u):
    ci, h, wo = u.shape
    ho = h // 2
    f32 = lambda v: jax.lax.bitcast_convert_type(v, jnp.float32)
    u4 = u.reshape(ci, ho, 2, wo)
    u0, u1 = u4[:, :, 0, :], u4[:, :, 1, :]
    r0c0, r0c1 = f32(u0 << 16), f32(u0 & jnp.uint32(0xFFFF0000))
    r1c0, r1c1 = f32(u1 << 16), f32(u1 & jnp.uint32(0xFFFF0000))
    pz = lambda a, dr, dc: jnp.pad(
        a, ((0, 1), (dr, 2 - dr), (dc, 128 - wo - dc)))
    buf = jnp.concatenate([
        pz(r1c1, 1, 1),
        pz(r1c0, 1, 0),
        pz(r0c1, 0, 1),
        pz(r0c0, 0, 0),
    ], axis=0)
    return buf.reshape(16, (ho + 2) * 128)


def _conv_image(u, w):
    flat = _phase_image(u)
    m = (flat.shape[1] // 128 - 2) * 128
    parts = [flat[:, ah * 128 + aw: ah * 128 + aw + m]
             for ah in (0, 1) for aw in (0, 1)]
    p = jnp.concatenate(parts, axis=0)
    return jnp.dot(w, p, preferred_element_type=jnp.float32)


def _fused_kernel(xu_ref, w_ref, g_ref, bt_ref, o_ref,
                  stat_vmem, sc_vmem, sh_vmem, *, ho, wo, imgs, inv_count, eps):
    phase = pl.program_id(0)
    i = pl.program_id(1)
    n_i = pl.num_programs(1)

    @pl.when(phase == 0)
    def _():
        @pl.when(i == 0)
        def _():
            stat_vmem[...] = jnp.zeros_like(stat_vmem)

        for img in range(imgs):
            y = _conv_image(xu_ref[img], w_ref[...])
            stat_vmem[0] += y
            stat_vmem[1] += y * y

        @pl.when(i == n_i - 1)
        def _():
            lane = jax.lax.broadcasted_iota(jnp.int32, (1, 1, 128), 2)
            maskf = (lane < wo).astype(jnp.float32)
            co = stat_vmem.shape[1]
            s3 = stat_vmem[0].reshape(co, ho, 128)
            q3 = stat_vmem[1].reshape(co, ho, 128)
            s = jnp.sum(jnp.sum(s3 * maskf, axis=2), axis=1, keepdims=True)
            q = jnp.sum(jnp.sum(q3 * maskf, axis=2), axis=1, keepdims=True)
            mean = s * inv_count
            var = q * inv_count - mean * mean
            sc = g_ref[...] * jax.lax.rsqrt(var + eps)
            sc_vmem[...] = sc
            sh_vmem[...] = bt_ref[...] - mean * sc

    @pl.when(phase == 1)
    def _():
        sc = sc_vmem[...]
        sh = sh_vmem[...]
        for img in range(imgs):
            y = _conv_image(xu_ref[img], w_ref[...])
            z = jnp.maximum(y * sc + sh, 0.0)
            o_ref[img] = z.reshape(z.shape[0], ho, 128)[:, :, :wo]


def kernel(x, w, b, gamma, beta):
    del b
    eps = 1e-5
    N, C_in, H, W = x.shape
    C_out = w.shape[0]
    Ho, Wo = H // 2, W // 2
    M = N * Ho * Wo

    x16 = x.astype(jnp.bfloat16).reshape(N, C_in, H, Wo, 2)
    xu = jax.lax.bitcast_convert_type(x16, jnp.uint32)

    w6 = w.reshape(C_out, C_in, 2, 2, 2, 2)
    w2 = w6.transpose(0, 2, 4, 3, 5, 1)
    w2 = jnp.pad(w2, ((0, 0),) * 5 + ((0, 4 - C_in),))
    w2 = w2.reshape(C_out, 64)

    IMG = 2 if N % 2 == 0 else 1
    vmem_limit = 100 << 20

    out = pl.pallas_call(
        functools.partial(_fused_kernel, ho=Ho, wo=Wo, imgs=IMG,
                          inv_count=float(1.0 / M), eps=float(eps)),
        out_shape=jax.ShapeDtypeStruct((N, C_out, Ho, Wo), jnp.float32),
        grid_spec=pltpu.PrefetchScalarGridSpec(
            num_scalar_prefetch=0,
            grid=(2, N // IMG),
            in_specs=[
                pl.BlockSpec((IMG, C_in, H, Wo), lambda p, i: (i, 0, 0, 0)),
                pl.BlockSpec((C_out, 64), lambda p, i: (0, 0)),
                pl.BlockSpec((C_out, 1), lambda p, i: (0, 0)),
                pl.BlockSpec((C_out, 1), lambda p, i: (0, 0)),
            ],
            out_specs=pl.BlockSpec((IMG, C_out, Ho, Wo),
                                   lambda p, i: (i * p, 0, 0, 0)),
            scratch_shapes=[pltpu.VMEM((2, C_out, Ho * 128), jnp.float32),
                            pltpu.VMEM((C_out, 1), jnp.float32),
                            pltpu.VMEM((C_out, 1), jnp.float32)]),
        compiler_params=pltpu.CompilerParams(
            dimension_semantics=("arbitrary", "arbitrary"),
            vmem_limit_bytes=vmem_limit),
    )(xu, w2, gamma.astype(jnp.float32).reshape(C_out, 1),
      beta.astype(jnp.float32).reshape(C_out, 1))

    return out
